# baseline (speedup 1.0000x reference)
import jax
import jax.numpy as jnp
from jax import lax
from jax.experimental import pallas as pl
from jax.experimental.pallas import tpu as pltpu
from jax.experimental.pallas import tpu_sc as plsc

ROWS = 16384
COLS = 2048
L = 16
GROUPS = COLS // L
NUM_WORKERS = 32
BLOCK_R = 8
NBUF_IN = 2
NBUF_OUT = 4
ROWS_PER_W = ROWS // NUM_WORKERS
NBLOCKS = ROWS_PER_W // BLOCK_R
assert NBLOCKS % NBUF_OUT == 0


def _sc_body(x_hbm, idx_hbm, out_hbm, idx_v,
             in0, in1, out0, out1, out2, out3,
             si0, si1, so0, so1, so2, so3):
    nc = 2
    wid = lax.axis_index("s") * nc + lax.axis_index("c")
    row_base = wid * ROWS_PER_W

    ins, isems = [in0, in1], [si0, si1]
    outs, osems = [out0, out1, out2, out3], [so0, so1, so2, so3]

    pltpu.sync_copy(idx_hbm, idx_v)

    def in_src(blk):
        return x_hbm.at[pl.ds(row_base + blk * BLOCK_R, BLOCK_R)]

    def out_dst(blk):
        return out_hbm.at[pl.ds(row_base + blk * BLOCK_R, BLOCK_R)]

    pltpu.async_copy(in_src(0), in0, si0)
    pltpu.async_copy(in_src(1), in1, si1)

    def quad_body(p, carry):
        for k in range(NBUF_OUT):
            blk = NBUF_OUT * p + k
            in_v = ins[k % NBUF_IN]
            out_v = outs[k]
            pltpu.make_async_copy(in_src(blk), in_v, isems[k % NBUF_IN]).wait()

            @pl.when(blk >= NBUF_OUT)
            def _wait_out():
                pltpu.make_async_copy(out_v, out_dst(blk - NBUF_OUT), osems[k]).wait()

            @plsc.parallel_loop(0, GROUPS, unroll=2)
            def g_body(g):
                col = idx_v[pl.ds(g * L, L)]
                vals = []
                for r in range(BLOCK_R):
                    row_splat = jnp.full((L,), r, jnp.int32)
                    vals.append(plsc.load_gather(in_v, [row_splat, col]))
                for r in range(BLOCK_R):
                    out_v[r, pl.ds(g * L, L)] = vals[r]

            pltpu.async_copy(out_v, out_dst(blk), osems[k])

            @pl.when(blk + NBUF_IN < NBLOCKS)
            def _next_in():
                pltpu.async_copy(in_src(blk + NBUF_IN), in_v, isems[k % NBUF_IN])
        return carry

    lax.fori_loop(0, NBLOCKS // NBUF_OUT, quad_body, 0)

    for k in range(NBUF_OUT):
        pltpu.make_async_copy(outs[k], out_dst(NBLOCKS - NBUF_OUT + k), osems[k]).wait()


def kernel(x, indices):
    mesh = plsc.VectorSubcoreMesh(core_axis_name="c", subcore_axis_name="s")
    f = pl.kernel(
        _sc_body,
        out_type=jax.ShapeDtypeStruct((ROWS, COLS), jnp.float32),
        mesh=mesh,
        compiler_params=pltpu.CompilerParams(needs_layout_passes=False),
        scratch_types=(
            [pltpu.VMEM((COLS,), jnp.int32)]
            + [pltpu.VMEM((BLOCK_R, COLS), jnp.float32) for _ in range(6)]
            + [pltpu.SemaphoreType.DMA for _ in range(6)]
        ),
    )
    return f(x, indices)


# speedup vs baseline: 1.0562x; 1.0562x over previous
import jax
import jax.numpy as jnp
from jax import lax
from jax.experimental import pallas as pl
from jax.experimental.pallas import tpu as pltpu
from jax.experimental.pallas import tpu_sc as plsc

ROWS = 16384
COLS = 2048
L = 16
GROUPS = COLS // L
NUM_WORKERS = 32
IN_R = 16
OUT_R = 8
ROWS_PER_W = ROWS // NUM_WORKERS
NIN = ROWS_PER_W // IN_R
NOUT = ROWS_PER_W // OUT_R


def _sc_body(x_hbm, idx_hbm, out_hbm,
             idx_v, in0, in1, out0, out1, si0, si1, so0, so1):
    nc = 2
    wid = lax.axis_index("s") * nc + lax.axis_index("c")
    row_base = wid * ROWS_PER_W

    ins, outs = [in0, in1], [out0, out1]
    isems, osems = [si0, si1], [so0, so1]

    pltpu.sync_copy(idx_hbm, idx_v)

    def in_src(ib):
        return x_hbm.at[pl.ds(row_base + ib * IN_R, IN_R)]

    def out_dst(ob):
        return out_hbm.at[pl.ds(row_base + ob * OUT_R, OUT_R)]

    pltpu.async_copy(in_src(0), in0, si0)
    pltpu.async_copy(in_src(1), in1, si1)

    def pair_body(p, carry):
        for k in range(2):
            ib = 2 * p + k
            in_v = ins[k]
            pltpu.make_async_copy(in_src(ib), in_v, isems[k]).wait()

            for h in range(2):
                ob = 2 * ib + h
                out_v = outs[h]
                @pl.when(ob >= 2)
                def _wait_out():
                    pltpu.make_async_copy(out_v, out_dst(ob - 2), osems[h]).wait()

                @plsc.parallel_loop(0, GROUPS, unroll=2)
                def g_body(g):
                    col = idx_v[pl.ds(g * L, L)]
                    vals = []
                    for r in range(OUT_R):
                        row_splat = jnp.full((L,), h * OUT_R + r, jnp.int32)
                        vals.append(plsc.load_gather(in_v, [row_splat, col]))
                    for r in range(OUT_R):
                        out_v[r, pl.ds(g * L, L)] = vals[r]

                pltpu.async_copy(out_v, out_dst(ob), osems[h])

            @pl.when(ib + 2 < NIN)
            def _next_in():
                pltpu.async_copy(in_src(ib + 2), in_v, isems[k])
        return carry

    lax.fori_loop(0, NIN // 2, pair_body, 0)

    pltpu.make_async_copy(out0, out_dst(NOUT - 2), so0).wait()
    pltpu.make_async_copy(out1, out_dst(NOUT - 1), so1).wait()


def kernel(x, indices):
    mesh = plsc.VectorSubcoreMesh(core_axis_name="c", subcore_axis_name="s")
    f = pl.kernel(
        _sc_body,
        out_type=jax.ShapeDtypeStruct((ROWS, COLS), jnp.float32),
        mesh=mesh,
        compiler_params=pltpu.CompilerParams(needs_layout_passes=False),
        scratch_types=[
            pltpu.VMEM((COLS,), jnp.int32),
            pltpu.VMEM((IN_R, COLS), jnp.float32),
            pltpu.VMEM((IN_R, COLS), jnp.float32),
            pltpu.VMEM((OUT_R, COLS), jnp.float32),
            pltpu.VMEM((OUT_R, COLS), jnp.float32),
            pltpu.SemaphoreType.DMA,
            pltpu.SemaphoreType.DMA,
            pltpu.SemaphoreType.DMA,
            pltpu.SemaphoreType.DMA,
        ],
    )
    return f(x, indices)


# speedup vs baseline: 1.0678x; 1.0110x over previous
import jax
import jax.numpy as jnp
from jax import lax
from jax.experimental import pallas as pl
from jax.experimental.pallas import tpu as pltpu
from jax.experimental.pallas import tpu_sc as plsc

ROWS = 16384
COLS = 2048
L = 16
GROUPS = COLS // L
NUM_WORKERS = 32
IN_R = 16
OUT_R = 8
ROWS_PER_W = ROWS // NUM_WORKERS
NIN = ROWS_PER_W // IN_R
NOUT = ROWS_PER_W // OUT_R


def _sc_body(x_hbm, idx_hbm, out_hbm,
             idx_v, in0, in1, out0, out1, si0, si1, so0, so1):
    nc = 2
    wid = lax.axis_index("s") * nc + lax.axis_index("c")
    row_base = wid * ROWS_PER_W

    ins, outs = [in0, in1], [out0, out1]
    isems, osems = [si0, si1], [so0, so1]

    pltpu.sync_copy(idx_hbm, idx_v)

    def in_src(ib):
        return x_hbm.at[pl.ds(row_base + ib * IN_R, IN_R)]

    def out_dst(ob):
        return out_hbm.at[pl.ds(row_base + ob * OUT_R, OUT_R)]

    pltpu.async_copy(in_src(0), in0, si0)
    pltpu.async_copy(in_src(1), in1, si1)

    def pair_body(p, carry):
        for k in range(2):
            ib = 2 * p + k
            in_v = ins[k]
            pltpu.make_async_copy(in_src(ib), in_v, isems[k]).wait()

            for h in range(2):
                ob = 2 * ib + h
                out_v = outs[h]
                @pl.when(ob >= 2)
                def _wait_out():
                    pltpu.make_async_copy(out_v, out_dst(ob - 2), osems[h]).wait()

                @plsc.parallel_loop(0, GROUPS, unroll=4)
                def g_body(g):
                    col = idx_v[pl.ds(g * L, L)]
                    vals = []
                    for r in range(OUT_R):
                        row_splat = jnp.full((L,), h * OUT_R + r, jnp.int32)
                        vals.append(plsc.load_gather(in_v, [row_splat, col]))
                    for r in range(OUT_R):
                        out_v[r, pl.ds(g * L, L)] = vals[r]

                pltpu.async_copy(out_v, out_dst(ob), osems[h])

            @pl.when(ib + 2 < NIN)
            def _next_in():
                pltpu.async_copy(in_src(ib + 2), in_v, isems[k])
        return carry

    lax.fori_loop(0, NIN // 2, pair_body, 0)

    pltpu.make_async_copy(out0, out_dst(NOUT - 2), so0).wait()
    pltpu.make_async_copy(out1, out_dst(NOUT - 1), so1).wait()


def kernel(x, indices):
    mesh = plsc.VectorSubcoreMesh(core_axis_name="c", subcore_axis_name="s")
    f = pl.kernel(
        _sc_body,
        out_type=jax.ShapeDtypeStruct((ROWS, COLS), jnp.float32),
        mesh=mesh,
        compiler_params=pltpu.CompilerParams(needs_layout_passes=False),
        scratch_types=[
            pltpu.VMEM((COLS,), jnp.int32),
            pltpu.VMEM((IN_R, COLS), jnp.float32),
            pltpu.VMEM((IN_R, COLS), jnp.float32),
            pltpu.VMEM((OUT_R, COLS), jnp.float32),
            pltpu.VMEM((OUT_R, COLS), jnp.float32),
            pltpu.SemaphoreType.DMA,
            pltpu.SemaphoreType.DMA,
            pltpu.SemaphoreType.DMA,
            pltpu.SemaphoreType.DMA,
        ],
    )
    return f(x, indices)
